# baseline (speedup 1.0000x reference)
import functools

import jax
import jax.numpy as jnp
from jax import lax
from jax.experimental import pallas as pl
from jax.experimental.pallas import tpu as pltpu
from jax.experimental.pallas import tpu_sc as plsc

B, C, H, W = 64, 2, 128, 128
HW = H * W
K = 500
KP = 512
L = 16
NC, NS = 2, 16
NW = NC * NS
BPW = B // NW
ROW = 4 * KP


def _make_sc_loss(interpret=False):
    mesh = plsc.VectorSubcoreMesh(core_axis_name="c", subcore_axis_name="s")

    @functools.partial(
        pl.kernel,
        out_type=jax.ShapeDtypeStruct((NW, 2 * L), jnp.float32),
        mesh=mesh,
        interpret=interpret,
        compiler_params=pltpu.CompilerParams(
            needs_layout_passes=False,
            skip_device_barrier=True,
            disable_bounds_checks=True,
            disable_semaphore_checks=True,
        ),
        scratch_types=[
            pltpu.VMEM((C, H, W), jnp.float32),
            pltpu.VMEM((C, H, W), jnp.float32),
            pltpu.VMEM((ROW,), jnp.float32),
            pltpu.VMEM((ROW,), jnp.float32),
            pltpu.VMEM((2 * L,), jnp.float32),
            pltpu.SemaphoreType.DMA,
            pltpu.SemaphoreType.DMA,
        ],
    )
    def sc_loss(feat_hbm, packed_hbm, out_hbm,
                feat0_v, feat1_v, row0_v, row1_v, stage_v, sem0, sem1):
        cid = lax.axis_index("c")
        sid = lax.axis_index("s")
        wid = sid * NC + cid
        b0 = wid * BPW

        cp0 = pltpu.async_copy(feat_hbm.at[b0], feat0_v, sem0)
        cp1 = pltpu.async_copy(feat_hbm.at[b0 + 1], feat1_v, sem1)
        pltpu.sync_copy(packed_hbm.at[b0], row0_v)
        pltpu.sync_copy(packed_hbm.at[b0 + 1], row1_v)

        zero = jnp.zeros((L,), jnp.int32)
        one = zero + 1
        num = jnp.zeros((L,), jnp.float32)
        msum = jnp.zeros((L,), jnp.float32)
        for cp, feat_v, row_v in ((cp0, feat0_v, row0_v),
                                  (cp1, feat1_v, row1_v)):
            cp.wait()
            for j in range(KP // L):
                s = j * L
                idx = plsc.bitcast(row_v[pl.ds(s, L)], jnp.int32)
                m = row_v[pl.ds(KP + s, L)]
                t0 = row_v[pl.ds(2 * KP + s, L)]
                t1 = row_v[pl.ds(3 * KP + s, L)]
                hi = lax.shift_right_logical(idx, 7)
                wi = lax.bitwise_and(idx, W - 1)
                x0 = plsc.load_gather(feat_v, [zero, hi, wi])
                x1 = plsc.load_gather(feat_v, [one, hi, wi])
                num = num + m * (jnp.abs(x0 - t0) + jnp.abs(x1 - t1))
                msum = msum + m

        stage_v[pl.ds(0, L)] = num
        stage_v[pl.ds(L, L)] = msum
        pltpu.sync_copy(stage_v, out_hbm.at[wid])

    return sc_loss


_SC_LOSS = _make_sc_loss()


def kernel(output, mask, ind, target):
    feat = output.astype(jnp.float32)
    pad = ((0, 0), (0, KP - K))
    packed = jnp.concatenate(
        [
            jnp.pad(lax.bitcast_convert_type(ind.astype(jnp.int32),
                                             jnp.float32), pad),
            jnp.pad(mask.astype(jnp.float32), pad),
            jnp.pad(target[:, :, 0].astype(jnp.float32), pad),
            jnp.pad(target[:, :, 1].astype(jnp.float32), pad),
        ],
        axis=1,
    )
    res = _SC_LOSS(feat, packed)
    num = jnp.sum(res[:, :L])
    msum = jnp.sum(res[:, L:])
    return num / (C * msum + 0.0001)


# speedup vs baseline: 1.0079x; 1.0079x over previous
import functools

import jax
import jax.numpy as jnp
from jax import lax
from jax.experimental import pallas as pl
from jax.experimental.pallas import tpu as pltpu
from jax.experimental.pallas import tpu_sc as plsc

B, C, H, W = 64, 2, 128, 128
HW = H * W
K = 500
KP = 512
L = 16
NC, NS = 2, 16
NW = NC * NS
BPW = B // NW
ROW = 4 * KP


def _make_sc_loss():
    mesh = plsc.VectorSubcoreMesh(core_axis_name="c", subcore_axis_name="s")

    @functools.partial(
        pl.kernel,
        out_type=jax.ShapeDtypeStruct((NW, 2 * L), jnp.float32),
        mesh=mesh,
        compiler_params=pltpu.CompilerParams(needs_layout_passes=False),
        scratch_types=[
            pltpu.VMEM((C, H, W), jnp.float32),
            pltpu.VMEM((C, H, W), jnp.float32),
            pltpu.VMEM((ROW,), jnp.float32),
            pltpu.VMEM((ROW,), jnp.float32),
            pltpu.VMEM((2 * L,), jnp.float32),
            pltpu.SemaphoreType.DMA,
            pltpu.SemaphoreType.DMA,
        ],
    )
    def sc_loss(feat_hbm, packed_hbm, out_hbm,
                feat0_v, feat1_v, row0_v, row1_v, stage_v, sem0, sem1):
        cid = lax.axis_index("c")
        sid = lax.axis_index("s")
        wid = sid * NC + cid
        b0 = wid * BPW

        cp0 = pltpu.async_copy(feat_hbm.at[b0], feat0_v, sem0)
        cp1 = pltpu.async_copy(feat_hbm.at[b0 + 1], feat1_v, sem1)
        pltpu.sync_copy(packed_hbm.at[b0], row0_v)
        pltpu.sync_copy(packed_hbm.at[b0 + 1], row1_v)

        zero = jnp.zeros((L,), jnp.int32)
        one = zero + 1
        num = jnp.zeros((L,), jnp.float32)
        msum = jnp.zeros((L,), jnp.float32)
        for cp, feat_v, row_v in ((cp0, feat0_v, row0_v),
                                  (cp1, feat1_v, row1_v)):
            cp.wait()
            for j in range(KP // L):
                s = j * L
                idx = plsc.bitcast(row_v[pl.ds(s, L)], jnp.int32)
                m = row_v[pl.ds(KP + s, L)]
                t0 = row_v[pl.ds(2 * KP + s, L)]
                t1 = row_v[pl.ds(3 * KP + s, L)]
                hi = lax.shift_right_logical(idx, 7)
                wi = lax.bitwise_and(idx, W - 1)
                x0 = plsc.load_gather(feat_v, [zero, hi, wi])
                x1 = plsc.load_gather(feat_v, [one, hi, wi])
                num = num + m * (jnp.abs(x0 - t0) + jnp.abs(x1 - t1))
                msum = msum + m

        stage_v[pl.ds(0, L)] = num
        stage_v[pl.ds(L, L)] = msum
        pltpu.sync_copy(stage_v, out_hbm.at[wid])

    return sc_loss


_SC_LOSS = _make_sc_loss()


def kernel(output, mask, ind, target):
    feat = output.astype(jnp.float32)
    pad = ((0, 0), (0, KP - K))
    packed = jnp.concatenate(
        [
            jnp.pad(lax.bitcast_convert_type(ind.astype(jnp.int32),
                                             jnp.float32), pad),
            jnp.pad(mask.astype(jnp.float32), pad),
            jnp.pad(target[:, :, 0].astype(jnp.float32), pad),
            jnp.pad(target[:, :, 1].astype(jnp.float32), pad),
        ],
        axis=1,
    )
    res = _SC_LOSS(feat, packed)
    num = jnp.sum(res[:, :L])
    msum = jnp.sum(res[:, L:])
    return num / (C * msum + 0.0001)
